# Initial kernel scaffold; baseline (speedup 1.0000x reference)
import jax
import jax.numpy as jnp
from jax.experimental import pallas as pl


def kernel(verts_attr, pix_to_face, bary_coords, face_tensor):
    raise NotImplementedError("write your pallas kernel here")


# speedup vs baseline: 22.2208x; 22.2208x over previous
import functools

import jax
import jax.numpy as jnp
from jax import lax
from jax.experimental import pallas as pl
from jax.experimental.pallas import tpu as pltpu
from jax.experimental.pallas import tpu_sc as plsc

L = 16


def _uv_body(NC, PPW, NB, P, verts_hbm, face_hbm, p2f_hbm, bary_hbm, out_hbm,
             verts_v, face_v, p2f_v, bary_v, out_v, sem):
    wid = lax.axis_index("s") * NC + lax.axis_index("c")
    base = wid * PPW

    copies = [
        pltpu.async_copy(verts_hbm, verts_v, sem),
        pltpu.async_copy(face_hbm, face_v, sem),
        pltpu.async_copy(p2f_hbm.at[pl.ds(base, PPW)], p2f_v, sem),
    ]
    for k in range(3):
        copies.append(pltpu.async_copy(
            bary_hbm.at[pl.ds(k * P + base, PPW)],
            bary_v.at[pl.ds(k * PPW, PPW)], sem))
    for c in copies:
        c.wait()

    iota = lax.iota(jnp.int32, L)

    def chunk(i, carry):
        off = i * L
        f3 = p2f_v[pl.ds(off, L)] * 3
        acc = [jnp.zeros((L,), jnp.float32) for _ in range(3)]
        for k in range(3):
            vk3 = plsc.load_gather(face_v, [f3 + k]) * 3
            bk = bary_v[pl.ds(k * PPW + off, L)]
            for d in range(3):
                acc[d] = acc[d] + bk * plsc.load_gather(verts_v, [vk3 + d])
        p3 = (off + iota) * 3
        for d in range(3):
            plsc.store_scatter(out_v, [p3 + d], acc[d])
        return carry

    lax.fori_loop(0, PPW // L, chunk, 0)

    outs = [pltpu.async_copy(out_v, out_hbm.at[b, pl.ds(base * 3, PPW * 3)], sem)
            for b in range(NB)]
    for c in outs:
        c.wait()


def kernel(verts_attr, pix_to_face, bary_coords, face_tensor):
    n, v, d = verts_attr.shape
    h, w = pix_to_face.shape
    P = h * w
    f = face_tensor.shape[0]

    info = plsc.get_sparse_core_info()
    NC, NS = info.num_cores, info.num_subcores
    NW = NC * NS
    PPW = P // NW

    vd = v * d
    vd_pad = (vd + 7) // 8 * 8
    verts_flat = jnp.pad(verts_attr[0].reshape(-1), (0, vd_pad - vd))
    face_flat = face_tensor.astype(jnp.int32).reshape(-1)
    p2f_flat = pix_to_face.astype(jnp.int32).reshape(-1)
    bary_t = bary_coords.reshape(P, 3).T.reshape(-1)

    mesh = plsc.VectorSubcoreMesh(core_axis_name="c", subcore_axis_name="s")
    body = functools.partial(_uv_body, NC, PPW, n, P)
    out = pl.kernel(
        body,
        out_type=jax.ShapeDtypeStruct((n, P * 3), jnp.float32),
        mesh=mesh,
        scratch_types=[
            pltpu.VMEM((vd_pad,), jnp.float32),
            pltpu.VMEM((3 * f,), jnp.int32),
            pltpu.VMEM((PPW,), jnp.int32),
            pltpu.VMEM((3 * PPW,), jnp.float32),
            pltpu.VMEM((3 * PPW,), jnp.float32),
            pltpu.SemaphoreType.DMA,
        ],
        compiler_params=pltpu.CompilerParams(needs_layout_passes=False),
    )(verts_flat, face_flat, p2f_flat, bary_t)
    return out.reshape(n, h, w, d)
